# baseline (speedup 1.0000x reference)
import jax
import jax.numpy as jnp
from jax import lax
from jax.experimental import pallas as pl
from jax.experimental.pallas import tpu as pltpu
from jax.experimental.pallas import tpu_sc as plsc

B = 16384
D = 64
W = 128
NC = 2
NS = 16
NW = NC * NS
BPW = B // NW
L = 16
CH = 128
NCH = BPW // CH
NG = CH // L


def _load_gather(ref, indices):
    return plsc.load_gather(ref, indices)


def _fm_body(u_hbm, i_hbm, j_hbm, eu_hbm, ei_hbm, ub_hbm, ib_hbm, b_hbm,
             out_i, out_j,
             uidx, iidx, jidx, pidx, urows, irows, jrows,
             ubl, ibl, jbl, bv, res_i, res_j, sem):
    wid = lax.axis_index("c") * NS + lax.axis_index("s")
    base = wid * BPW

    pltpu.sync_copy(u_hbm.at[pl.ds(base, BPW)], uidx)
    pltpu.sync_copy(i_hbm.at[pl.ds(base, BPW)], iidx)
    pltpu.sync_copy(j_hbm.at[pl.ds(base, BPW)], jidx)
    pltpu.sync_copy(b_hbm, bv)

    iota16 = lax.iota(jnp.int32, L)

    def pbody(q, _):
        s = pl.ds(q * L, L)
        pidx[0, s] = uidx[s] >> 1
        pidx[1, s] = iidx[s] >> 1
        pidx[2, s] = jidx[s] >> 1
        pidx[3, s] = uidx[s] >> 7
        pidx[4, s] = iidx[s] >> 7
        pidx[5, s] = jidx[s] >> 7
        return 0

    lax.fori_loop(0, BPW // L, pbody, 0)

    def chunk(k, _):
        cs = pl.ds(k * CH, CH)
        cps = [pltpu.async_copy(eu_hbm.at[pidx.at[0, cs]], urows, sem),
               pltpu.async_copy(ei_hbm.at[pidx.at[1, cs]], irows, sem),
               pltpu.async_copy(ei_hbm.at[pidx.at[2, cs]], jrows, sem),
               pltpu.async_copy(ub_hbm.at[pidx.at[3, cs]], ubl, sem),
               pltpu.async_copy(ib_hbm.at[pidx.at[4, cs]], ibl, sem),
               pltpu.async_copy(ib_hbm.at[pidx.at[5, cs]], jbl, sem)]
        for cp in cps:
            cp.wait()
        bias = bv[...]

        def gbody(g, _):
            s = pl.ds(k * CH + g * L, L)
            uoff = (uidx[s] & 1) * D
            ioff = (iidx[s] & 1) * D
            joff = (jidx[s] & 1) * D
            ids = g * L + iota16

            def fbody(f, carry):
                acc_i, acc_j = carry
                ue = _load_gather(urows, [ids, uoff + f])
                ie = _load_gather(irows, [ids, ioff + f])
                je = _load_gather(jrows, [ids, joff + f])
                return acc_i + ue * ie, acc_j + ue * je

            ub = _load_gather(ubl, [ids, uidx[s] & (W - 1)])
            bi = _load_gather(ibl, [ids, iidx[s] & (W - 1)])
            bj = _load_gather(jbl, [ids, jidx[s] & (W - 1)])
            acc0 = jnp.zeros((L,), jnp.float32)
            acc_i, acc_j = lax.fori_loop(0, D, fbody, (acc0, acc0), unroll=8)
            res_i[s] = acc_i + ub + bi + bias
            res_j[s] = acc_j + ub + bj + bias
            return 0

        lax.fori_loop(0, NG, gbody, 0)
        return 0

    lax.fori_loop(0, NCH, chunk, 0)

    pltpu.sync_copy(res_i, out_i.at[pl.ds(base, BPW)])
    pltpu.sync_copy(res_j, out_j.at[pl.ds(base, BPW)])


@jax.jit
def _pair_fm(u1, i1, j1, eu2, ei2, ub2, ib2, b16):
    mesh = plsc.VectorSubcoreMesh(core_axis_name="c", subcore_axis_name="s",
                                  num_cores=NC, num_subcores=NS)
    f = pl.kernel(
        _fm_body,
        out_type=[jax.ShapeDtypeStruct((B,), jnp.float32),
                  jax.ShapeDtypeStruct((B,), jnp.float32)],
        mesh=mesh,
        compiler_params=pltpu.CompilerParams(needs_layout_passes=False,
                                             use_tc_tiling_on_sc=True),
        scratch_types=[
            pltpu.VMEM((BPW,), jnp.int32),
            pltpu.VMEM((BPW,), jnp.int32),
            pltpu.VMEM((BPW,), jnp.int32),
            pltpu.VMEM((6, BPW), jnp.int32),
            pltpu.VMEM((CH, W), jnp.float32),
            pltpu.VMEM((CH, W), jnp.float32),
            pltpu.VMEM((CH, W), jnp.float32),
            pltpu.VMEM((CH, W), jnp.float32),
            pltpu.VMEM((CH, W), jnp.float32),
            pltpu.VMEM((CH, W), jnp.float32),
            pltpu.VMEM((L,), jnp.float32),
            pltpu.VMEM((BPW,), jnp.float32),
            pltpu.VMEM((BPW,), jnp.float32),
            pltpu.SemaphoreType.DMA,
        ],
    )
    return f(u1, i1, j1, eu2, ei2, ub2, ib2, b16)


def kernel(u, i, j, c, embed_user, embed_item, u_bias, i_bias, bias_):
    del c
    u1 = u.astype(jnp.int32)
    i1 = i.astype(jnp.int32)
    j1 = j.astype(jnp.int32)
    eu2 = embed_user.reshape(-1, W)
    ei2 = embed_item.reshape(-1, W)
    nu = u_bias.shape[0]
    ni = i_bias.shape[0]
    pu = (-nu) % W
    pi = (-ni) % W
    ub2 = jnp.pad(u_bias.reshape(-1), (0, pu)).reshape(-1, W)
    ib2 = jnp.pad(i_bias.reshape(-1), (0, pi)).reshape(-1, W)
    b16 = jnp.broadcast_to(bias_, (L,))
    return tuple(_pair_fm(u1, i1, j1, eu2, ei2, ub2, ib2, b16))
